# Initial kernel scaffold; baseline (speedup 1.0000x reference)
import jax
import jax.numpy as jnp
from jax.experimental import pallas as pl


def kernel(x, edge_index, W1l, W1r, b1, W2l, W2r, b2):
    raise NotImplementedError("write your pallas kernel here")


# speedup vs baseline: 6.1409x; 6.1409x over previous
import functools

import jax
import jax.numpy as jnp
from jax import lax
from jax.experimental import pallas as pl
from jax.experimental.pallas import tpu as pltpu
from jax.experimental.pallas import tpu_sc as plsc

NC = 2
NS = 16
CH = 128
ZR = 208
D = 128


@functools.lru_cache(maxsize=None)
def _make_segsum(n, e):
    per_w = e // (NC * NS)
    nch = per_w // CH
    tail = per_w - nch * CH
    rpt = (n // NS) // 8 * 8
    rem = n - NS * rpt
    assert e % (NC * NS) == 0 and rpt % ZR == 0 and 0 <= rem <= ZR
    assert per_w % 8 == 0 and tail % 8 == 0 and rem % 8 == 0

    mesh = plsc.VectorSubcoreMesh(core_axis_name="c", subcore_axis_name="s")

    scratch = [
        pltpu.VMEM((CH,), jnp.int32),
        pltpu.VMEM((CH,), jnp.int32),
        pltpu.VMEM((CH, D), jnp.float32),
        pltpu.VMEM((ZR, D), jnp.float32),
        pltpu.VMEM_SHARED((n, D), jnp.float32),
        pltpu.SemaphoreType.DMA,
    ]
    if tail:
        scratch += [
            pltpu.VMEM((tail,), jnp.int32),
            pltpu.VMEM((tail,), jnp.int32),
            pltpu.VMEM((tail, D), jnp.float32),
        ]

    @functools.partial(
        pl.kernel,
        out_type=jax.ShapeDtypeStruct((2 * n, D), jnp.float32),
        mesh=mesh,
        scratch_types=scratch,
    )
    def segsum(y_hbm, src_hbm, dst_hbm, out_hbm, src_v, dst_v, rows_v,
               zero_v, acc_sh, sem, *tail_refs):
        cid = lax.axis_index("c")
        sid = lax.axis_index("s")
        wid = cid * NS + sid

        def zfill(r, _):
            for j in range(D // 16):
                zero_v[r, pl.ds(j * 16, 16)] = jnp.zeros((16,), jnp.float32)
            return _
        lax.fori_loop(0, ZR, zfill, 0)
        zbase = sid * rpt

        def zcopy(t, _):
            pltpu.sync_copy(zero_v, acc_sh.at[pl.ds(zbase + t * ZR, ZR)])
            return _
        lax.fori_loop(0, rpt // ZR, zcopy, 0)
        if rem:
            @pl.when(sid == NS - 1)
            def _zrem():
                pltpu.sync_copy(zero_v.at[pl.ds(0, rem)],
                                acc_sh.at[pl.ds(NS * rpt, rem)])
        plsc.subcore_barrier()

        base_w = wid * per_w

        def chunk(t, _):
            base = base_w + t * CH
            pltpu.sync_copy(src_hbm.at[pl.ds(base, CH)], src_v)
            pltpu.sync_copy(dst_hbm.at[pl.ds(base, CH)], dst_v)
            pltpu.async_copy(y_hbm.at[src_v], rows_v, sem).wait()
            pltpu.sync_copy(rows_v, acc_sh.at[dst_v], add=True)
            return _
        lax.fori_loop(0, nch, chunk, 0)

        if tail:
            src_t, dst_t, rows_t = tail_refs
            tb = base_w + nch * CH
            pltpu.sync_copy(src_hbm.at[pl.ds(tb, tail)], src_t)
            pltpu.sync_copy(dst_hbm.at[pl.ds(tb, tail)], dst_t)
            pltpu.async_copy(y_hbm.at[src_t], rows_t, sem).wait()
            pltpu.sync_copy(rows_t, acc_sh.at[dst_t], add=True)

        plsc.subcore_barrier()
        pltpu.sync_copy(acc_sh.at[pl.ds(sid * rpt, rpt)],
                        out_hbm.at[pl.ds(cid * n + sid * rpt, rpt)])
        if rem:
            @pl.when(sid == NS - 1)
            def _orem():
                pltpu.sync_copy(acc_sh.at[pl.ds(NS * rpt, rem)],
                                out_hbm.at[pl.ds(cid * n + NS * rpt, rem)])

    return segsum


def _make_mid_body(d_hid):
    def _mid_body(p0_ref, p1_ref, x_ref, w1l_ref, w1r_ref, b1_ref, w2r_ref,
                  b2_ref, hp_ref, r2_ref):
        agg = p0_ref[...] + p1_ref[...]
        h = jnp.maximum(
            jnp.dot(agg, w1l_ref[...], preferred_element_type=jnp.float32)
            + b1_ref[...]
            + jnp.dot(x_ref[...], w1r_ref[...],
                      preferred_element_type=jnp.float32),
            0.0)
        hp_ref[...] = jnp.concatenate(
            [h, jnp.zeros_like(h)], axis=1)
        r2_ref[...] = (jnp.dot(h, w2r_ref[...],
                               preferred_element_type=jnp.float32)
                       + b2_ref[...])
    return _mid_body


def _make_out_body(d_hid):
    def _out_body(q0_ref, q1_ref, r2_ref, w2l_ref, o_ref):
        agg = (q0_ref[...] + q1_ref[...])[:, :d_hid]
        z = jnp.dot(agg, w2l_ref[...],
                    preferred_element_type=jnp.float32) + r2_ref[...]
        m = jnp.max(z, axis=1, keepdims=True)
        lse = m + jnp.log(jnp.sum(jnp.exp(z - m), axis=1, keepdims=True))
        o_ref[...] = z - lse
    return _out_body


def _row_spec(rb, cols, row_off=0):
    return pl.BlockSpec((rb, cols), lambda i, _o=row_off: (i + _o, 0))


def _full_spec(shape):
    return pl.BlockSpec(shape, lambda i: (0,) * len(shape))


@jax.jit
def kernel(x, edge_index, W1l, W1r, b1, W2l, W2r, b2):
    n, d_in = x.shape
    e = edge_index.shape[1]
    d_hid = W1l.shape[1]
    c = W2l.shape[1]
    assert d_in == D and 2 * d_hid == D

    src = edge_index[0]
    dst = edge_index[1]
    b1_r = b1.reshape(1, d_hid)
    b2_r = b2.reshape(1, c)

    rb = 1000
    grid = (n // rb,)
    segsum = _make_segsum(n, e)

    agg1 = segsum(x, src, dst)

    hp, r2 = pl.pallas_call(
        _make_mid_body(d_hid),
        grid=grid,
        in_specs=[_row_spec(rb, D), _row_spec(rb, D, n // rb),
                  _row_spec(rb, D), _full_spec((d_in, d_hid)),
                  _full_spec((d_in, d_hid)), _full_spec((1, d_hid)),
                  _full_spec((d_hid, c)), _full_spec((1, c))],
        out_specs=[_row_spec(rb, D), _row_spec(rb, c)],
        out_shape=[jax.ShapeDtypeStruct((n, D), jnp.float32),
                   jax.ShapeDtypeStruct((n, c), jnp.float32)],
    )(agg1, agg1, x, W1l, W1r, b1_r, W2r, b2_r)

    agg2 = segsum(hp, src, dst)

    out = pl.pallas_call(
        _make_out_body(d_hid),
        grid=grid,
        in_specs=[_row_spec(rb, D), _row_spec(rb, D, n // rb),
                  _row_spec(rb, c), _full_spec((d_hid, c))],
        out_specs=_row_spec(rb, c),
        out_shape=jax.ShapeDtypeStruct((n, c), jnp.float32),
    )(agg2, agg2, r2, W2l)

    return out


# speedup vs baseline: 11.4648x; 1.8670x over previous
import functools

import jax
import jax.numpy as jnp
from jax import lax
from jax.experimental import pallas as pl
from jax.experimental.pallas import tpu as pltpu
from jax.experimental.pallas import tpu_sc as plsc

NC = 2
NS = 16
CH = 128
ZR = 48
D = 128


NB = 2


@functools.lru_cache(maxsize=None)
def _make_segsum(n, e):
    tch = e // CH
    q = tch // (NC * NS)
    xr = tch - q * NC * NS
    ng = q // NB
    rpt = (n // NS) // 8 * 8
    rem = n - NS * rpt
    assert e % CH == 0 and q % NB == 0 and ng >= 2
    assert rpt % ZR == 0 and 0 <= rem <= ZR and rem % 8 == 0

    mesh = plsc.VectorSubcoreMesh(core_axis_name="c", subcore_axis_name="s")

    scratch = [
        pltpu.VMEM((NB, 2, CH), jnp.int32),
        pltpu.VMEM((NB, CH, D), jnp.float32),
        pltpu.VMEM((ZR, D), jnp.float32),
        pltpu.VMEM_SHARED((n, D), jnp.float32),
        pltpu.SemaphoreType.DMA,
        pltpu.SemaphoreType.DMA,
        pltpu.SemaphoreType.DMA,
        pltpu.SemaphoreType.DMA,
    ]

    @functools.partial(
        pl.kernel,
        out_type=jax.ShapeDtypeStruct((2 * n, D), jnp.float32),
        mesh=mesh,
        scratch_types=scratch,
    )
    def segsum(y_hbm, ec_hbm, out_hbm, idx_v, rows_v, zero_v, acc_sh,
               sg0, sg1, ss0, ss1):
        cid = lax.axis_index("c")
        sid = lax.axis_index("s")
        wid = cid * NS + sid
        semg = (sg0, sg1)
        sems = (ss0, ss1)

        def idx_load(t, b):
            pltpu.sync_copy(ec_hbm.at[t], idx_v.at[b])

        def gather_start(b):
            pltpu.async_copy(y_hbm.at[idx_v.at[b, 0]], rows_v.at[b], semg[b])

        def gather_wait(b):
            pltpu.make_async_copy(y_hbm.at[idx_v.at[b, 0]], rows_v.at[b],
                                  semg[b]).wait()

        def scatter_start(b):
            pltpu.async_copy(rows_v.at[b], acc_sh.at[idx_v.at[b, 1]],
                             sems[b], add=True)

        def scatter_wait(b):
            pltpu.make_async_copy(rows_v.at[b], acc_sh.at[idx_v.at[b, 1]],
                                  sems[b]).wait()

        def zfill(r, _):
            for j in range(D // 16):
                zero_v[r, pl.ds(j * 16, 16)] = jnp.zeros((16,), jnp.float32)
            return _
        lax.fori_loop(0, ZR, zfill, 0)
        zbase = sid * rpt

        def zcopy(t, _):
            pltpu.sync_copy(zero_v, acc_sh.at[pl.ds(zbase + t * ZR, ZR)])
            return _
        lax.fori_loop(0, rpt // ZR, zcopy, 0)
        if rem:
            @pl.when(sid == NS - 1)
            def _zrem():
                pltpu.sync_copy(zero_v.at[pl.ds(0, rem)],
                                acc_sh.at[pl.ds(NS * rpt, rem)])
        plsc.subcore_barrier()

        c0 = wid * q

        for b in range(NB):
            idx_load(c0 + b, b)
            gather_start(b)

        def group(t2, _):
            for b in range(NB):
                gather_wait(b)
                scatter_start(b)
                scatter_wait(b)
                idx_load(c0 + t2 * NB + b, b)
                gather_start(b)
            return _
        lax.fori_loop(1, ng, group, 0)

        for b in range(NB):
            gather_wait(b)
            scatter_start(b)
        for b in range(NB):
            scatter_wait(b)

        if xr:
            @pl.when(wid < xr)
            def _extra():
                idx_load(tch - xr + wid, 0)
                gather_start(0)
                gather_wait(0)
                scatter_start(0)
                scatter_wait(0)

        plsc.subcore_barrier()
        pltpu.sync_copy(acc_sh.at[pl.ds(sid * rpt, rpt)],
                        out_hbm.at[pl.ds(cid * n + sid * rpt, rpt)])
        if rem:
            @pl.when(sid == NS - 1)
            def _orem():
                pltpu.sync_copy(acc_sh.at[pl.ds(NS * rpt, rem)],
                                out_hbm.at[pl.ds(cid * n + NS * rpt, rem)])

    return segsum


def _make_mid_body(d_hid):
    def _mid_body(p0_ref, p1_ref, x_ref, w1l_ref, w1r_ref, b1_ref, w2r_ref,
                  b2_ref, hp_ref, r2_ref):
        agg = p0_ref[...] + p1_ref[...]
        h = jnp.maximum(
            jnp.dot(agg, w1l_ref[...], preferred_element_type=jnp.float32)
            + b1_ref[...]
            + jnp.dot(x_ref[...], w1r_ref[...],
                      preferred_element_type=jnp.float32),
            0.0)
        hp_ref[...] = jnp.concatenate(
            [h, jnp.zeros_like(h)], axis=1)
        r2_ref[...] = (jnp.dot(h, w2r_ref[...],
                               preferred_element_type=jnp.float32)
                       + b2_ref[...])
    return _mid_body


def _make_out_body(d_hid):
    def _out_body(q0_ref, q1_ref, r2_ref, w2l_ref, o_ref):
        agg = (q0_ref[...] + q1_ref[...])[:, :d_hid]
        z = jnp.dot(agg, w2l_ref[...],
                    preferred_element_type=jnp.float32) + r2_ref[...]
        m = jnp.max(z, axis=1, keepdims=True)
        lse = m + jnp.log(jnp.sum(jnp.exp(z - m), axis=1, keepdims=True))
        o_ref[...] = z - lse
    return _out_body


def _row_spec(rb, cols, row_off=0):
    return pl.BlockSpec((rb, cols), lambda i, _o=row_off: (i + _o, 0))


def _full_spec(shape):
    return pl.BlockSpec(shape, lambda i: (0,) * len(shape))


@jax.jit
def kernel(x, edge_index, W1l, W1r, b1, W2l, W2r, b2):
    n, d_in = x.shape
    e = edge_index.shape[1]
    d_hid = W1l.shape[1]
    c = W2l.shape[1]
    assert d_in == D and 2 * d_hid == D

    ec = edge_index.reshape(2, e // CH, CH).transpose(1, 0, 2)
    b1_r = b1.reshape(1, d_hid)
    b2_r = b2.reshape(1, c)

    rb = 1000
    grid = (n // rb,)
    segsum = _make_segsum(n, e)

    agg1 = segsum(x, ec)

    hp, r2 = pl.pallas_call(
        _make_mid_body(d_hid),
        grid=grid,
        in_specs=[_row_spec(rb, D), _row_spec(rb, D, n // rb),
                  _row_spec(rb, D), _full_spec((d_in, d_hid)),
                  _full_spec((d_in, d_hid)), _full_spec((1, d_hid)),
                  _full_spec((d_hid, c)), _full_spec((1, c))],
        out_specs=[_row_spec(rb, D), _row_spec(rb, c)],
        out_shape=[jax.ShapeDtypeStruct((n, D), jnp.float32),
                   jax.ShapeDtypeStruct((n, c), jnp.float32)],
    )(agg1, agg1, x, W1l, W1r, b1_r, W2r, b2_r)

    agg2 = segsum(hp, ec)

    out = pl.pallas_call(
        _make_out_body(d_hid),
        grid=grid,
        in_specs=[_row_spec(rb, D), _row_spec(rb, D, n // rb),
                  _row_spec(rb, c), _full_spec((d_hid, c))],
        out_specs=_row_spec(rb, c),
        out_shape=jax.ShapeDtypeStruct((n, c), jnp.float32),
    )(agg2, agg2, r2, W2l)

    return out


# speedup vs baseline: 11.9038x; 1.0383x over previous
import functools

import jax
import jax.numpy as jnp
from jax import lax
from jax.experimental import pallas as pl
from jax.experimental.pallas import tpu as pltpu
from jax.experimental.pallas import tpu_sc as plsc

NC = 2
NS = 16
CH = 128
ZR = 48
D = 128


NB = 3


@functools.lru_cache(maxsize=None)
def _make_segsum(n, e):
    tch = e // CH
    q = tch // (NC * NS)
    xr = tch - q * NC * NS
    rpt = (n // NS) // 8 * 8
    rem = n - NS * rpt
    zf = rpt // CH
    zr = rpt - zf * CH
    assert e % CH == 0 and q % NB == 0 and q >= 2 * NB
    assert 0 <= rem <= CH and rem % 8 == 0 and zr % 8 == 0

    mesh = plsc.VectorSubcoreMesh(core_axis_name="c", subcore_axis_name="s")

    scratch = [
        pltpu.VMEM((NB, 2, CH), jnp.int32),
        pltpu.VMEM((NB, CH, D), jnp.float32),
        pltpu.VMEM_SHARED((n, D), jnp.float32),
        pltpu.SemaphoreType.DMA,
        pltpu.SemaphoreType.DMA,
        pltpu.SemaphoreType.DMA,
        pltpu.SemaphoreType.DMA,
        pltpu.SemaphoreType.DMA,
        pltpu.SemaphoreType.DMA,
    ]

    @functools.partial(
        pl.kernel,
        out_type=jax.ShapeDtypeStruct((2 * n, D), jnp.float32),
        mesh=mesh,
        scratch_types=scratch,
    )
    def segsum(y_hbm, ec_hbm, out_hbm, idx_v, rows_v, acc_sh,
               sg0, sg1, sg2, ss0, ss1, ss2):
        cid = lax.axis_index("c")
        sid = lax.axis_index("s")
        wid = cid * NS + sid
        semg = (sg0, sg1, sg2)
        sems = (ss0, ss1, ss2)

        def idx_load(t, b):
            pltpu.sync_copy(ec_hbm.at[t], idx_v.at[b % NB])

        def gather_start(b):
            b = b % NB
            pltpu.async_copy(y_hbm.at[idx_v.at[b, 0]], rows_v.at[b], semg[b])

        def gather_wait(b):
            b = b % NB
            pltpu.make_async_copy(y_hbm.at[idx_v.at[b, 0]], rows_v.at[b],
                                  semg[b]).wait()

        def scatter_start(b):
            b = b % NB
            pltpu.async_copy(rows_v.at[b], acc_sh.at[idx_v.at[b, 1]],
                             sems[b], add=True)

        def scatter_wait(b):
            b = b % NB
            pltpu.make_async_copy(rows_v.at[b], acc_sh.at[idx_v.at[b, 1]],
                                  sems[b]).wait()

        def zfill(r, _):
            for j in range(D // 16):
                rows_v[0, r, pl.ds(j * 16, 16)] = jnp.zeros((16,),
                                                            jnp.float32)
            return _
        lax.fori_loop(0, CH, zfill, 0)
        zbase = sid * rpt

        def zcopy(t, _):
            pltpu.sync_copy(rows_v.at[0], acc_sh.at[pl.ds(zbase + t * CH,
                                                          CH)])
            return _
        lax.fori_loop(0, zf, zcopy, 0)
        if zr:
            pltpu.sync_copy(rows_v.at[0, pl.ds(0, zr)],
                            acc_sh.at[pl.ds(zbase + zf * CH, zr)])
        if rem:
            @pl.when(sid == NS - 1)
            def _zrem():
                pltpu.sync_copy(rows_v.at[0, pl.ds(0, rem)],
                                acc_sh.at[pl.ds(NS * rpt, rem)])
        plsc.subcore_barrier()

        c0 = wid * q

        def visit(t, k, swt=True, pref=True):
            if swt:
                scatter_wait(k - 1)
            if pref:
                idx_load(c0 + t + 2, k + 2)
                gather_start(k + 2)
            gather_wait(k)
            scatter_start(k)

        idx_load(c0 + 0, 0)
        gather_start(0)
        idx_load(c0 + 1, 1)
        gather_start(1)
        visit(0, 0, swt=False)
        visit(1, 1)
        visit(2, 2)

        def group(g, _):
            for k3 in range(NB):
                visit(NB + g * NB + k3, k3)
            return _
        lax.fori_loop(0, (q - 2 * NB) // NB, group, 0)

        visit(q - 3, q - 3)
        visit(q - 2, q - 2, pref=False)
        visit(q - 1, q - 1, pref=False)
        scatter_wait(q - 1)

        if xr:
            @pl.when(wid < xr)
            def _extra():
                idx_load(tch - xr + wid, 0)
                gather_start(0)
                gather_wait(0)
                scatter_start(0)
                scatter_wait(0)

        plsc.subcore_barrier()
        pltpu.sync_copy(acc_sh.at[pl.ds(sid * rpt, rpt)],
                        out_hbm.at[pl.ds(cid * n + sid * rpt, rpt)])
        if rem:
            @pl.when(sid == NS - 1)
            def _orem():
                pltpu.sync_copy(acc_sh.at[pl.ds(NS * rpt, rem)],
                                out_hbm.at[pl.ds(cid * n + NS * rpt, rem)])

    return segsum


def _make_mid_body(d_hid):
    def _mid_body(p0_ref, p1_ref, x_ref, w1l_ref, w1r_ref, b1_ref, w2r_ref,
                  b2_ref, hp_ref, r2_ref):
        agg = p0_ref[...] + p1_ref[...]
        h = jnp.maximum(
            jnp.dot(agg, w1l_ref[...], preferred_element_type=jnp.float32)
            + b1_ref[...]
            + jnp.dot(x_ref[...], w1r_ref[...],
                      preferred_element_type=jnp.float32),
            0.0)
        hp_ref[...] = jnp.concatenate(
            [h, jnp.zeros_like(h)], axis=1)
        r2_ref[...] = (jnp.dot(h, w2r_ref[...],
                               preferred_element_type=jnp.float32)
                       + b2_ref[...])
    return _mid_body


def _make_out_body(d_hid):
    def _out_body(q0_ref, q1_ref, r2_ref, w2l_ref, o_ref):
        agg = (q0_ref[...] + q1_ref[...])[:, :d_hid]
        z = jnp.dot(agg, w2l_ref[...],
                    preferred_element_type=jnp.float32) + r2_ref[...]
        m = jnp.max(z, axis=1, keepdims=True)
        lse = m + jnp.log(jnp.sum(jnp.exp(z - m), axis=1, keepdims=True))
        o_ref[...] = z - lse
    return _out_body


def _row_spec(rb, cols, row_off=0):
    return pl.BlockSpec((rb, cols), lambda i, _o=row_off: (i + _o, 0))


def _full_spec(shape):
    return pl.BlockSpec(shape, lambda i: (0,) * len(shape))


@jax.jit
def kernel(x, edge_index, W1l, W1r, b1, W2l, W2r, b2):
    n, d_in = x.shape
    e = edge_index.shape[1]
    d_hid = W1l.shape[1]
    c = W2l.shape[1]
    assert d_in == D and 2 * d_hid == D

    ec = edge_index.reshape(2, e // CH, CH).transpose(1, 0, 2)
    b1_r = b1.reshape(1, d_hid)
    b2_r = b2.reshape(1, c)

    rb = 1000
    grid = (n // rb,)
    segsum = _make_segsum(n, e)

    agg1 = segsum(x, ec)

    hp, r2 = pl.pallas_call(
        _make_mid_body(d_hid),
        grid=grid,
        in_specs=[_row_spec(rb, D), _row_spec(rb, D, n // rb),
                  _row_spec(rb, D), _full_spec((d_in, d_hid)),
                  _full_spec((d_in, d_hid)), _full_spec((1, d_hid)),
                  _full_spec((d_hid, c)), _full_spec((1, c))],
        out_specs=[_row_spec(rb, D), _row_spec(rb, c)],
        out_shape=[jax.ShapeDtypeStruct((n, D), jnp.float32),
                   jax.ShapeDtypeStruct((n, c), jnp.float32)],
    )(agg1, agg1, x, W1l, W1r, b1_r, W2r, b2_r)

    agg2 = segsum(hp, ec)

    out = pl.pallas_call(
        _make_out_body(d_hid),
        grid=grid,
        in_specs=[_row_spec(rb, D), _row_spec(rb, D, n // rb),
                  _row_spec(rb, c), _full_spec((d_hid, c))],
        out_specs=_row_spec(rb, c),
        out_shape=jax.ShapeDtypeStruct((n, c), jnp.float32),
    )(agg2, agg2, r2, W2l)

    return out


# speedup vs baseline: 11.9452x; 1.0035x over previous
import functools

import jax
import jax.numpy as jnp
from jax import lax
from jax.experimental import pallas as pl
from jax.experimental.pallas import tpu as pltpu
from jax.experimental.pallas import tpu_sc as plsc

NC = 2
NS = 16
CH = 128
ZR = 48
D = 128


NB = 3


@functools.lru_cache(maxsize=None)
def _make_segsum(n, e, d):
    tch = e // CH
    q = tch // (NC * NS)
    xr = tch - q * NC * NS
    rpt = (n // NS) // 8 * 8
    rem = n - NS * rpt
    zf = rpt // CH
    zr = rpt - zf * CH
    assert e % CH == 0 and q % NB == 0 and q >= 2 * NB
    assert 0 <= rem <= CH and rem % 8 == 0 and zr % 8 == 0

    mesh = plsc.VectorSubcoreMesh(core_axis_name="c", subcore_axis_name="s")

    scratch = [
        pltpu.VMEM((NB, 2, CH), jnp.int32),
        pltpu.VMEM((NB, CH, d), jnp.float32),
        pltpu.VMEM_SHARED((n, d), jnp.float32),
        pltpu.SemaphoreType.DMA,
        pltpu.SemaphoreType.DMA,
        pltpu.SemaphoreType.DMA,
        pltpu.SemaphoreType.DMA,
        pltpu.SemaphoreType.DMA,
        pltpu.SemaphoreType.DMA,
    ]

    @functools.partial(
        pl.kernel,
        out_type=jax.ShapeDtypeStruct((2 * n, d), jnp.float32),
        mesh=mesh,
        scratch_types=scratch,
    )
    def segsum(y_hbm, ec_hbm, out_hbm, idx_v, rows_v, acc_sh,
               sg0, sg1, sg2, ss0, ss1, ss2):
        cid = lax.axis_index("c")
        sid = lax.axis_index("s")
        wid = cid * NS + sid
        semg = (sg0, sg1, sg2)
        sems = (ss0, ss1, ss2)
        tbl = y_hbm

        def idx_load(t, b):
            pltpu.sync_copy(ec_hbm.at[t], idx_v.at[b % NB])

        def gather_start(b):
            b = b % NB
            pltpu.async_copy(tbl.at[idx_v.at[b, 0]], rows_v.at[b], semg[b])

        def gather_wait(b):
            b = b % NB
            pltpu.make_async_copy(tbl.at[idx_v.at[b, 0]], rows_v.at[b],
                                  semg[b]).wait()

        def scatter_start(b):
            b = b % NB
            pltpu.async_copy(rows_v.at[b], acc_sh.at[idx_v.at[b, 1]],
                             sems[b], add=True)

        def scatter_wait(b):
            b = b % NB
            pltpu.make_async_copy(rows_v.at[b], acc_sh.at[idx_v.at[b, 1]],
                                  sems[b]).wait()

        def zfill(r, _):
            for j in range(d // 16):
                rows_v[0, r, pl.ds(j * 16, 16)] = jnp.zeros((16,),
                                                            jnp.float32)
            return _
        lax.fori_loop(0, CH, zfill, 0)
        zbase = sid * rpt

        def zcopy(t, _):
            pltpu.sync_copy(rows_v.at[0], acc_sh.at[pl.ds(zbase + t * CH,
                                                          CH)])
            return _
        lax.fori_loop(0, zf, zcopy, 0)
        if zr:
            pltpu.sync_copy(rows_v.at[0, pl.ds(0, zr)],
                            acc_sh.at[pl.ds(zbase + zf * CH, zr)])
        if rem:
            @pl.when(sid == NS - 1)
            def _zrem():
                pltpu.sync_copy(rows_v.at[0, pl.ds(0, rem)],
                                acc_sh.at[pl.ds(NS * rpt, rem)])
        plsc.subcore_barrier()

        c0 = wid * q

        def visit(t, k, swt=True, pref=True):
            if swt:
                scatter_wait(k - 1)
            if pref:
                idx_load(c0 + t + 2, k + 2)
                gather_start(k + 2)
            gather_wait(k)
            scatter_start(k)

        idx_load(c0 + 0, 0)
        gather_start(0)
        idx_load(c0 + 1, 1)
        gather_start(1)
        visit(0, 0, swt=False)
        visit(1, 1)
        visit(2, 2)

        def group(g, _):
            for k3 in range(NB):
                visit(NB + g * NB + k3, k3)
            return _
        lax.fori_loop(0, (q - 2 * NB) // NB, group, 0)

        visit(q - 3, q - 3)
        visit(q - 2, q - 2, pref=False)
        visit(q - 1, q - 1, pref=False)
        scatter_wait(q - 1)

        if xr:
            @pl.when(wid < xr)
            def _extra():
                idx_load(tch - xr + wid, 0)
                gather_start(0)
                gather_wait(0)
                scatter_start(0)
                scatter_wait(0)

        plsc.subcore_barrier()
        pltpu.sync_copy(acc_sh.at[pl.ds(sid * rpt, rpt)],
                        out_hbm.at[pl.ds(cid * n + sid * rpt, rpt)])
        if rem:
            @pl.when(sid == NS - 1)
            def _orem():
                pltpu.sync_copy(acc_sh.at[pl.ds(NS * rpt, rem)],
                                out_hbm.at[pl.ds(cid * n + NS * rpt, rem)])

    return segsum


def _lin_body(x_ref, w_ref, b_ref, o_ref):
    o_ref[...] = (jnp.dot(x_ref[...], w_ref[...],
                          preferred_element_type=jnp.float32) + b_ref[...])


def _make_mid_body(d_hid):
    def _mid_body(p0_ref, p1_ref, xr1_ref, w1l_ref, hp_ref):
        agg = p0_ref[...] + p1_ref[...]
        h = jnp.maximum(
            jnp.dot(agg, w1l_ref[...], preferred_element_type=jnp.float32)
            + xr1_ref[...],
            0.0)
        hp_ref[...] = jnp.concatenate([h, jnp.zeros_like(h)], axis=1)
    return _mid_body


def _make_r2_body(d_hid):
    def _r2_body(hp_ref, w2r_ref, b2_ref, o_ref):
        o_ref[...] = (jnp.dot(hp_ref[...][:, :d_hid], w2r_ref[...],
                              preferred_element_type=jnp.float32)
                      + b2_ref[...])
    return _r2_body


def _make_out_body(d_hid):
    def _out_body(q0_ref, q1_ref, r2_ref, w2l_ref, o_ref):
        agg = (q0_ref[...] + q1_ref[...])[:, :d_hid]
        z = jnp.dot(agg, w2l_ref[...],
                    preferred_element_type=jnp.float32) + r2_ref[...]
        m = jnp.max(z, axis=1, keepdims=True)
        lse = m + jnp.log(jnp.sum(jnp.exp(z - m), axis=1, keepdims=True))
        o_ref[...] = z - lse
    return _out_body


def _row_spec(rb, cols, row_off=0):
    return pl.BlockSpec((rb, cols), lambda i, _o=row_off: (i + _o, 0))


def _full_spec(shape):
    return pl.BlockSpec(shape, lambda i: (0,) * len(shape))


@jax.jit
def kernel(x, edge_index, W1l, W1r, b1, W2l, W2r, b2):
    n, d_in = x.shape
    e = edge_index.shape[1]
    d_hid = W1l.shape[1]
    c = W2l.shape[1]
    assert d_in == D and 2 * d_hid == D

    ec = edge_index.reshape(2, e // CH, CH).transpose(1, 0, 2)
    b1_r = b1.reshape(1, d_hid)
    b2_r = b2.reshape(1, c)

    rb = 1000
    grid = (n // rb,)

    segsum = _make_segsum(n, e, D)

    agg1 = segsum(x, ec)

    xr1 = pl.pallas_call(
        _lin_body,
        grid=grid,
        in_specs=[_row_spec(rb, D), _full_spec((d_in, d_hid)),
                  _full_spec((1, d_hid))],
        out_specs=_row_spec(rb, d_hid),
        out_shape=jax.ShapeDtypeStruct((n, d_hid), jnp.float32),
    )(x, W1r, b1_r)

    hp = pl.pallas_call(
        _make_mid_body(d_hid),
        grid=grid,
        in_specs=[_row_spec(rb, D), _row_spec(rb, D, n // rb),
                  _row_spec(rb, d_hid), _full_spec((d_in, d_hid))],
        out_specs=_row_spec(rb, D),
        out_shape=jax.ShapeDtypeStruct((n, D), jnp.float32),
    )(agg1, agg1, xr1, W1l)

    agg2 = segsum(hp, ec)

    r2 = pl.pallas_call(
        _make_r2_body(d_hid),
        grid=grid,
        in_specs=[_row_spec(rb, D), _full_spec((d_hid, c)),
                  _full_spec((1, c))],
        out_specs=_row_spec(rb, c),
        out_shape=jax.ShapeDtypeStruct((n, c), jnp.float32),
    )(hp, W2r, b2_r)

    out = pl.pallas_call(
        _make_out_body(d_hid),
        grid=grid,
        in_specs=[_row_spec(rb, D), _row_spec(rb, D, n // rb),
                  _row_spec(rb, c), _full_spec((d_hid, c))],
        out_specs=_row_spec(rb, c),
        out_shape=jax.ShapeDtypeStruct((n, c), jnp.float32),
    )(agg2, agg2, r2, W2l)

    return out


# speedup vs baseline: 12.1799x; 1.0197x over previous
import functools

import jax
import jax.numpy as jnp
from jax import lax
from jax.experimental import pallas as pl
from jax.experimental.pallas import tpu as pltpu
from jax.experimental.pallas import tpu_sc as plsc

NC = 2
NS = 16
CH = 128
ZR = 48
D = 128


NB = 3


@functools.lru_cache(maxsize=None)
def _make_segsum(n, e, d):
    tch = e // CH
    q = tch // (NC * NS)
    xr = tch - q * NC * NS
    rpt = (n // NS) // 8 * 8
    rem = n - NS * rpt
    zf = rpt // CH
    zr = rpt - zf * CH
    assert e % CH == 0 and q % NB == 0 and q >= 2 * NB
    assert 0 <= rem <= CH and rem % 8 == 0 and zr % 8 == 0

    mesh = plsc.VectorSubcoreMesh(core_axis_name="c", subcore_axis_name="s")

    scratch = [
        pltpu.VMEM((NB, 2, CH), jnp.int32),
        pltpu.VMEM((NB, CH, d), jnp.float32),
        pltpu.VMEM_SHARED((n, d), jnp.float32),
        pltpu.SemaphoreType.DMA,
        pltpu.SemaphoreType.DMA,
        pltpu.SemaphoreType.DMA,
        pltpu.SemaphoreType.DMA,
        pltpu.SemaphoreType.DMA,
        pltpu.SemaphoreType.DMA,
    ]

    @functools.partial(
        pl.kernel,
        out_type=jax.ShapeDtypeStruct((2 * n, d), jnp.float32),
        mesh=mesh,
        scratch_types=scratch,
    )
    def segsum(y_hbm, ec_hbm, out_hbm, idx_v, rows_v, acc_sh,
               sg0, sg1, sg2, ss0, ss1, ss2):
        cid = lax.axis_index("c")
        sid = lax.axis_index("s")
        wid = cid * NS + sid
        semg = (sg0, sg1, sg2)
        sems = (ss0, ss1, ss2)
        tbl = y_hbm

        def idx_load(t, b):
            pltpu.sync_copy(ec_hbm.at[t], idx_v.at[b % NB])

        def gather_start(b):
            b = b % NB
            pltpu.async_copy(tbl.at[idx_v.at[b, 0]], rows_v.at[b], semg[b])

        def gather_wait(b):
            b = b % NB
            pltpu.make_async_copy(tbl.at[idx_v.at[b, 0]], rows_v.at[b],
                                  semg[b]).wait()

        def scatter_start(b):
            b = b % NB
            pltpu.async_copy(rows_v.at[b], acc_sh.at[idx_v.at[b, 1]],
                             sems[b], add=True)

        def scatter_wait(b):
            b = b % NB
            pltpu.make_async_copy(rows_v.at[b], acc_sh.at[idx_v.at[b, 1]],
                                  sems[b]).wait()

        def zfill(r, _):
            for j in range(d // 16):
                rows_v[0, r, pl.ds(j * 16, 16)] = jnp.zeros((16,),
                                                            jnp.float32)
            return _
        lax.fori_loop(0, CH, zfill, 0)
        zbase = sid * rpt

        def zcopy(t, _):
            pltpu.sync_copy(rows_v.at[0], acc_sh.at[pl.ds(zbase + t * CH,
                                                          CH)])
            return _
        lax.fori_loop(0, zf, zcopy, 0)
        if zr:
            pltpu.sync_copy(rows_v.at[0, pl.ds(0, zr)],
                            acc_sh.at[pl.ds(zbase + zf * CH, zr)])
        if rem:
            @pl.when(sid == NS - 1)
            def _zrem():
                pltpu.sync_copy(rows_v.at[0, pl.ds(0, rem)],
                                acc_sh.at[pl.ds(NS * rpt, rem)])
        plsc.subcore_barrier()

        c0 = wid * q

        def visit(t, k, swt=True, pref=True):
            if swt:
                scatter_wait(k - 1)
            if pref:
                idx_load(c0 + t + 2, k + 2)
                gather_start(k + 2)
            gather_wait(k)
            scatter_start(k)

        idx_load(c0 + 0, 0)
        gather_start(0)
        idx_load(c0 + 1, 1)
        gather_start(1)
        visit(0, 0, swt=False)
        visit(1, 1)
        visit(2, 2)

        def group(g, _):
            for k3 in range(NB):
                visit(NB + g * NB + k3, k3)
            return _
        lax.fori_loop(0, (q - 2 * NB) // NB, group, 0)

        visit(q - 3, q - 3)
        visit(q - 2, q - 2, pref=False)
        visit(q - 1, q - 1, pref=False)
        scatter_wait(q - 1)

        if xr:
            @pl.when(wid < xr)
            def _extra():
                idx_load(tch - xr + wid, 0)
                gather_start(0)
                gather_wait(0)
                scatter_start(0)
                scatter_wait(0)

        plsc.subcore_barrier()
        pltpu.sync_copy(acc_sh.at[pl.ds(sid * rpt, rpt)],
                        out_hbm.at[pl.ds(cid * n + sid * rpt, rpt)])
        if rem:
            @pl.when(sid == NS - 1)
            def _orem():
                pltpu.sync_copy(acc_sh.at[pl.ds(NS * rpt, rem)],
                                out_hbm.at[pl.ds(cid * n + NS * rpt, rem)])

    return segsum


def _lin_body(x_ref, w_ref, b_ref, o_ref):
    o_ref[...] = (jnp.dot(x_ref[...], w_ref[...],
                          preferred_element_type=jnp.float32) + b_ref[...])


def _make_mid_body(d_hid):
    def _mid_body(p0_ref, p1_ref, xr1_ref, w1l_ref, hp_ref):
        agg = p0_ref[...] + p1_ref[...]
        h = jnp.maximum(
            jnp.dot(agg, w1l_ref[...], preferred_element_type=jnp.float32)
            + xr1_ref[...],
            0.0)
        hp_ref[...] = jnp.concatenate([h, jnp.zeros_like(h)], axis=1)
    return _mid_body


def _make_r2_body(d_hid):
    def _r2_body(hp_ref, w2r_ref, b2_ref, o_ref):
        o_ref[...] = (jnp.dot(hp_ref[...][:, :d_hid], w2r_ref[...],
                              preferred_element_type=jnp.float32)
                      + b2_ref[...])
    return _r2_body


def _make_out_body(d_hid):
    def _out_body(q0_ref, q1_ref, r2_ref, w2l_ref, o_ref):
        agg = (q0_ref[...] + q1_ref[...])[:, :d_hid]
        z = jnp.dot(agg, w2l_ref[...],
                    preferred_element_type=jnp.float32) + r2_ref[...]
        m = jnp.max(z, axis=1, keepdims=True)
        lse = m + jnp.log(jnp.sum(jnp.exp(z - m), axis=1, keepdims=True))
        o_ref[...] = z - lse
    return _out_body


def _row_spec(rb, cols, row_off=0):
    return pl.BlockSpec((rb, cols), lambda i, _o=row_off: (i + _o, 0))


def _full_spec(shape):
    return pl.BlockSpec(shape, lambda i: (0,) * len(shape))


@jax.jit
def kernel(x, edge_index, W1l, W1r, b1, W2l, W2r, b2):
    n, d_in = x.shape
    e = edge_index.shape[1]
    d_hid = W1l.shape[1]
    c = W2l.shape[1]
    assert d_in == D and 2 * d_hid == D

    ec = edge_index.reshape(2, e // CH, CH).transpose(1, 0, 2)
    b1_r = b1.reshape(1, d_hid)
    b2_r = b2.reshape(1, c)

    rb = 2000
    grid = (n // rb,)

    segsum = _make_segsum(n, e, D)

    agg1 = segsum(x, ec)

    xr1 = pl.pallas_call(
        _lin_body,
        grid=grid,
        in_specs=[_row_spec(rb, D), _full_spec((d_in, d_hid)),
                  _full_spec((1, d_hid))],
        out_specs=_row_spec(rb, d_hid),
        out_shape=jax.ShapeDtypeStruct((n, d_hid), jnp.float32),
    )(x, W1r, b1_r)

    hp = pl.pallas_call(
        _make_mid_body(d_hid),
        grid=grid,
        in_specs=[_row_spec(rb, D), _row_spec(rb, D, n // rb),
                  _row_spec(rb, d_hid), _full_spec((d_in, d_hid))],
        out_specs=_row_spec(rb, D),
        out_shape=jax.ShapeDtypeStruct((n, D), jnp.float32),
    )(agg1, agg1, xr1, W1l)

    agg2 = segsum(hp, ec)

    r2 = pl.pallas_call(
        _make_r2_body(d_hid),
        grid=grid,
        in_specs=[_row_spec(rb, D), _full_spec((d_hid, c)),
                  _full_spec((1, c))],
        out_specs=_row_spec(rb, c),
        out_shape=jax.ShapeDtypeStruct((n, c), jnp.float32),
    )(hp, W2r, b2_r)

    out = pl.pallas_call(
        _make_out_body(d_hid),
        grid=grid,
        in_specs=[_row_spec(rb, D), _row_spec(rb, D, n // rb),
                  _row_spec(rb, c), _full_spec((d_hid, c))],
        out_specs=_row_spec(rb, c),
        out_shape=jax.ShapeDtypeStruct((n, c), jnp.float32),
    )(agg2, agg2, r2, W2l)

    return out
